# baseline (speedup 1.0000x reference)
import functools
import math

import jax
import jax.numpy as jnp
from jax.experimental import pallas as pl
from jax.experimental.pallas import tpu as pltpu

H = 16
W_BLK = 128


def _fused_attn_kernel(x_ref, wqt_ref, wkt_ref, wvt_ref, wot_ref, out_ref,
                       *, nb, inv_scale):
    h = pl.program_id(0)
    xv = x_ref[...]
    q = jnp.dot(xv, wqt_ref[...], preferred_element_type=jnp.float32)
    k = jnp.dot(xv, wkt_ref[...], preferred_element_type=jnp.float32)
    v = jnp.dot(xv, wvt_ref[...], preferred_element_type=jnp.float32)
    wot = wot_ref[...]
    for j in range(nb):
        sl = slice(j * W_BLK, (j + 1) * W_BLK)
        qj = q[sl, :]
        kj = k[sl, :]
        vj = v[sl, :]
        s = jax.lax.dot_general(qj, kj, (((1,), (1,)), ((), ())),
                                preferred_element_type=jnp.float32)
        s = s * inv_scale
        s = s - jnp.max(s, axis=-1, keepdims=True)
        p = jnp.exp(s)
        p = p / jnp.sum(p, axis=-1, keepdims=True)
        oj = jnp.dot(p, vj, preferred_element_type=jnp.float32)
        contrib = jnp.dot(oj, wot, preferred_element_type=jnp.float32)

        @pl.when(h == 0)
        def _():
            out_ref[sl, :] = contrib

        @pl.when(h != 0)
        def _():
            out_ref[sl, :] += contrib


def kernel(x, Wq, Wk, Wv, Wo):
    B_, T_, D_ = x.shape
    Dh = D_ // H
    N = B_ * T_
    nb = N // W_BLK
    inv_scale = 1.0 / math.sqrt(Dh)

    x2 = x.reshape(N, D_)
    WqT = Wq.T
    WkT = Wk.T
    WvT = Wv.T
    WoT = Wo.T

    body = functools.partial(_fused_attn_kernel, nb=nb, inv_scale=inv_scale)
    out = pl.pallas_call(
        body,
        grid=(H,),
        in_specs=[
            pl.BlockSpec((N, D_), lambda h: (0, 0)),
            pl.BlockSpec((D_, Dh), lambda h: (0, h)),
            pl.BlockSpec((D_, Dh), lambda h: (0, h)),
            pl.BlockSpec((D_, Dh), lambda h: (0, h)),
            pl.BlockSpec((Dh, D_), lambda h: (h, 0)),
        ],
        out_specs=pl.BlockSpec((N, D_), lambda h: (0, 0)),
        out_shape=jax.ShapeDtypeStruct((N, D_), jnp.float32),
        compiler_params=pltpu.CompilerParams(
            dimension_semantics=("arbitrary",),
        ),
    )(x2, WqT, WkT, WvT, WoT)
    return out.reshape(B_, T_, D_)


# speedup vs baseline: 1.0364x; 1.0364x over previous
import functools
import math

import jax
import jax.numpy as jnp
from jax.experimental import pallas as pl
from jax.experimental.pallas import tpu as pltpu

H = 16
W_BLK = 128


def _fused_attn_kernel(x_ref, wqt_ref, wkt_ref, wvt_ref, wot_ref, out_ref,
                       *, nb, inv_scale):
    h = pl.program_id(0)
    xv = x_ref[...]
    q = jnp.dot(xv, wqt_ref[...], preferred_element_type=jnp.float32)
    k = jnp.dot(xv, wkt_ref[...], preferred_element_type=jnp.float32)
    v = jnp.dot(xv, wvt_ref[...], preferred_element_type=jnp.float32)
    wot = wot_ref[...]
    for j in range(nb):
        sl = slice(j * W_BLK, (j + 1) * W_BLK)
        qj = q[sl, :].astype(jnp.bfloat16)
        kj = k[sl, :].astype(jnp.bfloat16)
        vj = v[sl, :].astype(jnp.bfloat16)
        s = jax.lax.dot_general(qj, kj, (((1,), (1,)), ((), ())),
                                preferred_element_type=jnp.float32)
        s = s * inv_scale
        s = s - jnp.max(s, axis=-1, keepdims=True)
        p = jnp.exp(s)
        p = p / jnp.sum(p, axis=-1, keepdims=True)
        oj = jnp.dot(p.astype(jnp.bfloat16), vj,
                     preferred_element_type=jnp.float32)
        contrib = jnp.dot(oj.astype(jnp.bfloat16), wot,
                          preferred_element_type=jnp.float32)

        @pl.when(h == 0)
        def _():
            out_ref[sl, :] = contrib

        @pl.when(h != 0)
        def _():
            out_ref[sl, :] += contrib


def kernel(x, Wq, Wk, Wv, Wo):
    B_, T_, D_ = x.shape
    Dh = D_ // H
    N = B_ * T_
    nb = N // W_BLK
    inv_scale = 1.0 / math.sqrt(Dh)

    x2 = x.reshape(N, D_).astype(jnp.bfloat16)
    WqT = Wq.T.astype(jnp.bfloat16)
    WkT = Wk.T.astype(jnp.bfloat16)
    WvT = Wv.T.astype(jnp.bfloat16)
    WoT = Wo.T.astype(jnp.bfloat16)

    body = functools.partial(_fused_attn_kernel, nb=nb, inv_scale=inv_scale)
    out = pl.pallas_call(
        body,
        grid=(H,),
        in_specs=[
            pl.BlockSpec((N, D_), lambda h: (0, 0)),
            pl.BlockSpec((D_, Dh), lambda h: (0, h)),
            pl.BlockSpec((D_, Dh), lambda h: (0, h)),
            pl.BlockSpec((D_, Dh), lambda h: (0, h)),
            pl.BlockSpec((Dh, D_), lambda h: (h, 0)),
        ],
        out_specs=pl.BlockSpec((N, D_), lambda h: (0, 0)),
        out_shape=jax.ShapeDtypeStruct((N, D_), jnp.float32),
        compiler_params=pltpu.CompilerParams(
            dimension_semantics=("arbitrary",),
        ),
    )(x2, WqT, WkT, WvT, WoT)
    return out.reshape(B_, T_, D_)


# speedup vs baseline: 2.0103x; 1.9397x over previous
import functools
import math

import jax
import jax.numpy as jnp
from jax.experimental import pallas as pl
from jax.experimental.pallas import tpu as pltpu

H = 16
W_BLK = 128
TOK = 256


def _fused_attn_kernel(x_ref, wqt_ref, wkt_ref, wvt_ref, wot_ref, out_ref,
                       o_scr, *, inv_scale):
    xv = x_ref[...]
    q = jnp.dot(xv, wqt_ref[...], preferred_element_type=jnp.float32)
    k = jnp.dot(xv, wkt_ref[...], preferred_element_type=jnp.float32)
    v = jnp.dot(xv, wvt_ref[...], preferred_element_type=jnp.float32)
    for h in range(H):
        cs = slice(h * W_BLK, (h + 1) * W_BLK)
        qh = q[:, cs].astype(jnp.bfloat16)
        kh = k[:, cs].astype(jnp.bfloat16)
        vh = v[:, cs].astype(jnp.bfloat16)
        for j in range(TOK // W_BLK):
            rs = slice(j * W_BLK, (j + 1) * W_BLK)
            s = jax.lax.dot_general(qh[rs, :], kh[rs, :],
                                    (((1,), (1,)), ((), ())),
                                    preferred_element_type=jnp.float32)
            s = s * inv_scale
            s = s - jnp.max(s, axis=-1, keepdims=True)
            p = jnp.exp(s)
            p = p / jnp.sum(p, axis=-1, keepdims=True)
            o_scr[rs, cs] = jnp.dot(p.astype(jnp.bfloat16), vh[rs, :],
                                    preferred_element_type=jnp.float32
                                    ).astype(jnp.bfloat16)
    out_ref[...] = jnp.dot(o_scr[...], wot_ref[...],
                           preferred_element_type=jnp.float32)


def kernel(x, Wq, Wk, Wv, Wo):
    B_, T_, D_ = x.shape
    N = B_ * T_
    Dh = D_ // H
    inv_scale = 1.0 / math.sqrt(Dh)

    x2 = x.reshape(N, D_).astype(jnp.bfloat16)
    WqT = Wq.T.astype(jnp.bfloat16)
    WkT = Wk.T.astype(jnp.bfloat16)
    WvT = Wv.T.astype(jnp.bfloat16)
    WoT = Wo.T.astype(jnp.bfloat16)

    body = functools.partial(_fused_attn_kernel, inv_scale=inv_scale)
    out = pl.pallas_call(
        body,
        grid=(N // TOK,),
        in_specs=[
            pl.BlockSpec((TOK, D_), lambda i: (i, 0)),
            pl.BlockSpec((D_, D_), lambda i: (0, 0)),
            pl.BlockSpec((D_, D_), lambda i: (0, 0)),
            pl.BlockSpec((D_, D_), lambda i: (0, 0)),
            pl.BlockSpec((D_, D_), lambda i: (0, 0)),
        ],
        out_specs=pl.BlockSpec((TOK, D_), lambda i: (i, 0)),
        out_shape=jax.ShapeDtypeStruct((N, D_), jnp.float32),
        scratch_shapes=[pltpu.VMEM((TOK, D_), jnp.bfloat16)],
        compiler_params=pltpu.CompilerParams(
            dimension_semantics=("parallel",),
        ),
    )(x2, WqT, WkT, WvT, WoT)
    return out.reshape(B_, T_, D_)


# speedup vs baseline: 2.5533x; 1.2701x over previous
import math

import jax
import jax.numpy as jnp
from jax.experimental import pallas as pl
from jax.experimental.pallas import tpu as pltpu

H = 16
W_BLK = 128
TOK = 256
NSUB = TOK // W_BLK


def _fused_attn_kernel(x_ref, wqt_ref, wkt_ref, wvt_ref, wot_ref, out_ref,
                       s_scr, o_scr):
    xv = x_ref[...]
    q = jnp.dot(xv, wqt_ref[...], preferred_element_type=jnp.float32)
    k = jnp.dot(xv, wkt_ref[...], preferred_element_type=jnp.float32)
    v = jnp.dot(xv, wvt_ref[...], preferred_element_type=jnp.float32)

    for h in range(H):
        cs = slice(h * W_BLK, (h + 1) * W_BLK)
        qh = q[:, cs]
        kh = k[:, cs]
        for j in range(NSUB):
            rs = slice(j * W_BLK, (j + 1) * W_BLK)
            b = h * NSUB + j
            s_scr[b * W_BLK:(b + 1) * W_BLK, :] = jax.lax.dot_general(
                qh[rs, :], kh[rs, :], (((1,), (1,)), ((), ())),
                preferred_element_type=jnp.float32)

    sv = s_scr[...]
    sv = sv - jnp.max(sv, axis=-1, keepdims=True)
    p = jnp.exp(sv)
    p = p / jnp.sum(p, axis=-1, keepdims=True)

    for h in range(H):
        cs = slice(h * W_BLK, (h + 1) * W_BLK)
        vh = v[:, cs]
        for j in range(NSUB):
            rs = slice(j * W_BLK, (j + 1) * W_BLK)
            b = h * NSUB + j
            o_scr[rs, cs] = jnp.dot(
                p[b * W_BLK:(b + 1) * W_BLK, :], vh[rs, :],
                preferred_element_type=jnp.float32).astype(jnp.bfloat16)

    out_ref[...] = jnp.dot(o_scr[...], wot_ref[...],
                           preferred_element_type=jnp.float32)


def kernel(x, Wq, Wk, Wv, Wo):
    B_, T_, D_ = x.shape
    N = B_ * T_
    Dh = D_ // H
    inv_scale = 1.0 / math.sqrt(Dh)

    x2 = x.reshape(N, D_).astype(jnp.bfloat16)
    WqT = (Wq.T * inv_scale).astype(jnp.bfloat16)
    WkT = Wk.T.astype(jnp.bfloat16)
    WvT = Wv.T.astype(jnp.bfloat16)
    WoT = Wo.T.astype(jnp.bfloat16)

    out = pl.pallas_call(
        _fused_attn_kernel,
        grid=(N // TOK,),
        in_specs=[
            pl.BlockSpec((TOK, D_), lambda i: (i, 0)),
            pl.BlockSpec((D_, D_), lambda i: (0, 0)),
            pl.BlockSpec((D_, D_), lambda i: (0, 0)),
            pl.BlockSpec((D_, D_), lambda i: (0, 0)),
            pl.BlockSpec((D_, D_), lambda i: (0, 0)),
        ],
        out_specs=pl.BlockSpec((TOK, D_), lambda i: (i, 0)),
        out_shape=jax.ShapeDtypeStruct((N, D_), jnp.float32),
        scratch_shapes=[
            pltpu.VMEM((H * NSUB * W_BLK, W_BLK), jnp.float32),
            pltpu.VMEM((TOK, D_), jnp.bfloat16),
        ],
        compiler_params=pltpu.CompilerParams(
            dimension_semantics=("parallel",),
        ),
    )(x2, WqT, WkT, WvT, WoT)
    return out.reshape(B_, T_, D_)


# speedup vs baseline: 3.4662x; 1.3575x over previous
import math

import jax
import jax.numpy as jnp
from jax.experimental import pallas as pl
from jax.experimental.pallas import tpu as pltpu

H = 16
W_BLK = 128
TOK = 256
NSUB = TOK // W_BLK

_TRANS = (((1,), (1,)), ((), ()))


def _fused_attn_kernel(x_ref, wq_ref, wk_ref, wv_ref, wo_ref, out_ref,
                       s_scr, o_scr, *, inv_scale):
    xv = x_ref[...].astype(jnp.bfloat16)
    q = jax.lax.dot_general(xv, wq_ref[...], _TRANS,
                            preferred_element_type=jnp.float32)
    k = jax.lax.dot_general(xv, wk_ref[...], _TRANS,
                            preferred_element_type=jnp.float32)
    v = jax.lax.dot_general(xv, wv_ref[...], _TRANS,
                            preferred_element_type=jnp.float32)

    for h in range(H):
        cs = slice(h * W_BLK, (h + 1) * W_BLK)
        qh = q[:, cs]
        kh = k[:, cs]
        for j in range(NSUB):
            rs = slice(j * W_BLK, (j + 1) * W_BLK)
            b = h * NSUB + j
            s_scr[b * W_BLK:(b + 1) * W_BLK, :] = jax.lax.dot_general(
                qh[rs, :], kh[rs, :], _TRANS,
                preferred_element_type=jnp.float32)

    sv = s_scr[...]
    sv = (sv - jnp.max(sv, axis=-1, keepdims=True)) * inv_scale
    p = jnp.exp(sv)
    p = p / jnp.sum(p, axis=-1, keepdims=True)

    for h in range(H):
        cs = slice(h * W_BLK, (h + 1) * W_BLK)
        vh = v[:, cs]
        for j in range(NSUB):
            rs = slice(j * W_BLK, (j + 1) * W_BLK)
            b = h * NSUB + j
            o_scr[rs, cs] = jnp.dot(
                p[b * W_BLK:(b + 1) * W_BLK, :], vh[rs, :],
                preferred_element_type=jnp.float32).astype(jnp.bfloat16)

    out_ref[...] = jax.lax.dot_general(o_scr[...], wo_ref[...], _TRANS,
                                       preferred_element_type=jnp.float32)


def kernel(x, Wq, Wk, Wv, Wo):
    B_, T_, D_ = x.shape
    N = B_ * T_
    Dh = D_ // H
    inv_scale = 1.0 / math.sqrt(Dh)

    x2 = x.reshape(N, D_)
    import functools
    body = functools.partial(_fused_attn_kernel, inv_scale=inv_scale)
    out = pl.pallas_call(
        body,
        grid=(N // TOK,),
        in_specs=[
            pl.BlockSpec((TOK, D_), lambda i: (i, 0)),
            pl.BlockSpec((D_, D_), lambda i: (0, 0)),
            pl.BlockSpec((D_, D_), lambda i: (0, 0)),
            pl.BlockSpec((D_, D_), lambda i: (0, 0)),
            pl.BlockSpec((D_, D_), lambda i: (0, 0)),
        ],
        out_specs=pl.BlockSpec((TOK, D_), lambda i: (i, 0)),
        out_shape=jax.ShapeDtypeStruct((N, D_), jnp.float32),
        scratch_shapes=[
            pltpu.VMEM((H * NSUB * W_BLK, W_BLK), jnp.float32),
            pltpu.VMEM((TOK, D_), jnp.bfloat16),
        ],
        compiler_params=pltpu.CompilerParams(
            dimension_semantics=("parallel",),
        ),
    )(x2, Wq.astype(jnp.bfloat16), Wk.astype(jnp.bfloat16),
      Wv.astype(jnp.bfloat16), Wo.astype(jnp.bfloat16))
    return out.reshape(B_, T_, D_)


# speedup vs baseline: 4.1758x; 1.2047x over previous
import functools
import math

import jax
import jax.numpy as jnp
from jax.experimental import pallas as pl
from jax.experimental.pallas import tpu as pltpu

H = 16
W_BLK = 128
TOK = 256
NSUB = TOK // W_BLK
CVT_ROWS = 512

_TRANS = (((1,), (1,)), ((), ()))


def _fused_attn_kernel(x_ref, wq_hbm, wk_hbm, wv_hbm, wo_hbm, out_ref,
                       wq_s, wk_s, wv_s, wo_s, stg, s_scr, o_scr, sems,
                       *, inv_scale, d):
    i = pl.program_id(0)
    nch = d // CVT_ROWS
    srcs = (wq_hbm, wk_hbm, wv_hbm, wo_hbm)
    dsts = (wq_s, wk_s, wv_s, wo_s)
    ntot = 4 * nch

    @pl.when(i == 0)
    def _convert():
        def dma(t, buf):
            w, c = divmod(t, nch)
            return pltpu.make_async_copy(
                srcs[w].at[pl.ds(c * CVT_ROWS, CVT_ROWS), :],
                stg.at[buf], sems.at[buf])

        dma(0, 0).start()
        for t in range(ntot):
            buf = t % 2
            if t + 1 < ntot:
                dma(t + 1, 1 - buf).start()
            dma(t, buf).wait()
            w, c = divmod(t, nch)
            dsts[w][c * CVT_ROWS:(c + 1) * CVT_ROWS, :] = (
                stg[buf].astype(jnp.bfloat16))

    xv = x_ref[...].astype(jnp.bfloat16)
    q = jax.lax.dot_general(xv, wq_s[...], _TRANS,
                            preferred_element_type=jnp.float32)
    k = jax.lax.dot_general(xv, wk_s[...], _TRANS,
                            preferred_element_type=jnp.float32)
    v = jax.lax.dot_general(xv, wv_s[...], _TRANS,
                            preferred_element_type=jnp.float32)

    for h in range(H):
        cs = slice(h * W_BLK, (h + 1) * W_BLK)
        qh = q[:, cs]
        kh = k[:, cs]
        for j in range(NSUB):
            rs = slice(j * W_BLK, (j + 1) * W_BLK)
            b = h * NSUB + j
            s_scr[b * W_BLK:(b + 1) * W_BLK, :] = jax.lax.dot_general(
                qh[rs, :], kh[rs, :], _TRANS,
                preferred_element_type=jnp.float32)

    sv = s_scr[...]
    sv = (sv - jnp.max(sv, axis=-1, keepdims=True)) * inv_scale
    p = jnp.exp(sv)
    p = p / jnp.sum(p, axis=-1, keepdims=True)

    for h in range(H):
        cs = slice(h * W_BLK, (h + 1) * W_BLK)
        vh = v[:, cs]
        for j in range(NSUB):
            rs = slice(j * W_BLK, (j + 1) * W_BLK)
            b = h * NSUB + j
            o_scr[rs, cs] = jnp.dot(
                p[b * W_BLK:(b + 1) * W_BLK, :], vh[rs, :],
                preferred_element_type=jnp.float32).astype(jnp.bfloat16)

    out_ref[...] = jax.lax.dot_general(o_scr[...], wo_s[...], _TRANS,
                                       preferred_element_type=jnp.float32)


def kernel(x, Wq, Wk, Wv, Wo):
    B_, T_, D_ = x.shape
    N = B_ * T_
    Dh = D_ // H
    inv_scale = 1.0 / math.sqrt(Dh)

    x2 = x.reshape(N, D_)
    body = functools.partial(_fused_attn_kernel, inv_scale=inv_scale, d=D_)
    out = pl.pallas_call(
        body,
        grid=(N // TOK,),
        in_specs=[
            pl.BlockSpec((TOK, D_), lambda i: (i, 0)),
            pl.BlockSpec(memory_space=pl.ANY),
            pl.BlockSpec(memory_space=pl.ANY),
            pl.BlockSpec(memory_space=pl.ANY),
            pl.BlockSpec(memory_space=pl.ANY),
        ],
        out_specs=pl.BlockSpec((TOK, D_), lambda i: (i, 0)),
        out_shape=jax.ShapeDtypeStruct((N, D_), jnp.float32),
        scratch_shapes=[
            pltpu.VMEM((D_, D_), jnp.bfloat16),
            pltpu.VMEM((D_, D_), jnp.bfloat16),
            pltpu.VMEM((D_, D_), jnp.bfloat16),
            pltpu.VMEM((D_, D_), jnp.bfloat16),
            pltpu.VMEM((2, CVT_ROWS, D_), jnp.float32),
            pltpu.VMEM((H * NSUB * W_BLK, W_BLK), jnp.float32),
            pltpu.VMEM((TOK, D_), jnp.bfloat16),
            pltpu.SemaphoreType.DMA((2,)),
        ],
        compiler_params=pltpu.CompilerParams(
            dimension_semantics=("arbitrary",),
        ),
    )(x2, Wq, Wk, Wv, Wo)
    return out.reshape(B_, T_, D_)
